# baseline (speedup 1.0000x reference)
import functools

import jax
import jax.numpy as jnp
from jax import lax
from jax.experimental import pallas as pl
from jax.experimental.pallas import tpu as pltpu
from jax.experimental.pallas import tpu_sc as plsc

H = 4
DH = 32
EXW = 16
CH = 96
E_PP = 160000
E_LP = 64000
N_NODES = 10000
N_PAD = 10112
RPT = 632


def _mm_body(x_ref, w_ref, o_ref):
    o_ref[...] = jnp.dot(x_ref[...], w_ref[...],
                         preferred_element_type=jnp.float32)


def _mm(x, w, block_rows):
    M, K = x.shape
    _, N = w.shape
    assert M % block_rows == 0
    return pl.pallas_call(
        _mm_body,
        grid=(M // block_rows,),
        in_specs=[pl.BlockSpec((block_rows, K), lambda i: (i, 0)),
                  pl.BlockSpec((K, N), lambda i: (0, 0))],
        out_specs=pl.BlockSpec((block_rows, N), lambda i: (i, 0)),
        out_shape=jax.ShapeDtypeStruct((M, N), jnp.float32),
    )(x, w)


GCH = 128


def _sc_gather(pw0, dst0, ll, src1, pw1, dst1):
    mesh = plsc.VectorSubcoreMesh(core_axis_name="c", subcore_axis_name="s")

    @functools.partial(
        pl.kernel,
        mesh=mesh,
        compiler_params=pltpu.CompilerParams(use_tc_tiling_on_sc=False),
        out_type=[jax.ShapeDtypeStruct((E_PP, 128), jnp.float32),
                  jax.ShapeDtypeStruct((E_LP, 128), jnp.float32),
                  jax.ShapeDtypeStruct((E_LP, 128), jnp.float32),
                  jax.ShapeDtypeStruct((E_LP, 128), jnp.float32)],
        scratch_types=[
            pltpu.VMEM((GCH,), jnp.int32),
            pltpu.VMEM((GCH,), jnp.int32),
            pltpu.VMEM((GCH, 128), jnp.float32),
            pltpu.VMEM((GCH, 128), jnp.float32),
            pltpu.VMEM((GCH, 256), jnp.float32),
            pltpu.VMEM((GCH, 256), jnp.float32),
            pltpu.SemaphoreType.DMA,
            pltpu.SemaphoreType.DMA,
        ],
    )
    def k(pw0_hbm, dst0_hbm, ll_hbm, src1_hbm, pw1_hbm, dst1_hbm,
          gw0_hbm, gsw_hbm, gsv_hbm, gd_hbm,
          idx_a, idx_b, n_a, n_b, w_a, w_b, sem_a, sem_b):
        c = lax.axis_index("c")
        s = lax.axis_index("s")
        wid = c * 16 + s

        def gseg(idx_hbm, table_hbm, outs, base, nfull, tail, buf_a, buf_b):
            wide = len(outs) > 1

            def wr(buf, o, n):
                for out_hbm, lo in outs:
                    srcb = (buf.at[pl.ds(0, n), pl.ds(lo, 128)]
                            if (wide or n != GCH) else buf)
                    pltpu.sync_copy(srcb, out_hbm.at[pl.ds(o, n)])

            npairs = nfull // 2
            pltpu.sync_copy(idx_hbm.at[pl.ds(base, GCH)], idx_a)
            pltpu.async_copy(table_hbm.at[idx_a], buf_a, sem_a)

            def body(j, _):
                o0 = base + (2 * j) * GCH
                o1 = base + (2 * j + 1) * GCH
                pltpu.sync_copy(idx_hbm.at[pl.ds(o1, GCH)], idx_b)
                pltpu.async_copy(table_hbm.at[idx_b], buf_b, sem_b)
                pltpu.make_async_copy(table_hbm.at[idx_a], buf_a, sem_a).wait()
                wr(buf_a, o0, GCH)

                @pl.when(j < npairs - 1)
                def _():
                    o2 = base + (2 * j + 2) * GCH
                    pltpu.sync_copy(idx_hbm.at[pl.ds(o2, GCH)], idx_a)
                    pltpu.async_copy(table_hbm.at[idx_a], buf_a, sem_a)

                pltpu.make_async_copy(table_hbm.at[idx_b], buf_b, sem_b).wait()
                wr(buf_b, o1, GCH)
                return 0
            lax.fori_loop(0, npairs, body, 0)

            tails = ([(base + 2 * npairs * GCH, GCH)] if nfull % 2 else [])
            tails.append((base + nfull * GCH, tail))
            for t, n in tails:
                it = idx_a.at[pl.ds(0, n)]
                bt = buf_a.at[pl.ds(0, n)]
                pltpu.sync_copy(idx_hbm.at[pl.ds(t, n)], it)
                pltpu.async_copy(table_hbm.at[it], bt, sem_a)
                pltpu.make_async_copy(table_hbm.at[it], bt, sem_a).wait()
                wr(buf_a, t, n)

        base0 = wid * (E_PP // 32)
        base1 = wid * (E_LP // 32)
        gseg(dst0_hbm, pw0_hbm, [(gw0_hbm, 0)], base0, 39, 8, n_a, n_b)
        gseg(src1_hbm, ll_hbm, [(gsw_hbm, 0), (gsv_hbm, 128)],
             base1, 15, 80, w_a, w_b)
        gseg(dst1_hbm, pw1_hbm, [(gd_hbm, 0)], base1, 15, 80, n_a, n_b)

    return k(pw0, dst0, ll, src1, pw1, dst1)


def _iota2(shape, d0, d1, div):
    return (jax.lax.broadcasted_iota(jnp.int32, shape, d0) // div
            == jax.lax.broadcasted_iota(jnp.int32, shape, d1)
            ).astype(jnp.float32)


def _edge0_body(f_ref, w_ref, g_ref, a_ref, o_ref, e_ref):
    x = jnp.dot(f_ref[...], w_ref[...], preferred_element_type=jnp.float32)
    pre = x[:, :128] + g_ref[...]
    hh = jnp.where(pre >= 0, pre, 0.2 * pre)
    sel = _iota2((H * DH, H), 0, 1, DH)
    ex = jnp.exp(jnp.dot(hh * a_ref[...], sel,
                         preferred_element_type=jnp.float32))
    rexp = _iota2((H, H * DH), 1, 0, DH)
    b = ex.shape[0]
    o_ref[...] = jnp.dot(ex, rexp, preferred_element_type=jnp.float32) * x[:, 128:]
    e_ref[...] = jnp.concatenate(
        [ex, jnp.zeros((b, EXW - H), jnp.float32)], axis=1)


def _edge1_body(f_ref, w_ref, gw_ref, gv_ref, gd_ref, a_ref, o_ref, e_ref):
    x = jnp.dot(f_ref[...], w_ref[...], preferred_element_type=jnp.float32)
    pre = x[:, :128] + gw_ref[...] + gd_ref[...]
    fv = x[:, 128:] + gv_ref[...]
    hh = jnp.where(pre >= 0, pre, 0.2 * pre)
    sel = _iota2((H * DH, H), 0, 1, DH)
    ex = jnp.exp(jnp.dot(hh * a_ref[...], sel,
                         preferred_element_type=jnp.float32))
    rexp = _iota2((H, H * DH), 1, 0, DH)
    b = ex.shape[0]
    o_ref[...] = jnp.dot(ex, rexp, preferred_element_type=jnp.float32) * fv
    e_ref[...] = jnp.concatenate(
        [jnp.zeros((b, H), jnp.float32), ex,
         jnp.zeros((b, EXW - 2 * H), jnp.float32)], axis=1)


def _edge0(f, w, g, a0f, block_rows=1000):
    E, K = f.shape
    row = lambda i: (i, 0)
    return pl.pallas_call(
        _edge0_body,
        grid=(E // block_rows,),
        in_specs=[pl.BlockSpec((block_rows, K), row),
                  pl.BlockSpec((K, 256), lambda i: (0, 0)),
                  pl.BlockSpec((block_rows, 128), row),
                  pl.BlockSpec((1, 128), lambda i: (0, 0))],
        out_specs=[pl.BlockSpec((block_rows, 128), row),
                   pl.BlockSpec((block_rows, EXW), row)],
        out_shape=[jax.ShapeDtypeStruct((E, 128), jnp.float32),
                   jax.ShapeDtypeStruct((E, EXW), jnp.float32)],
    )(f, w, g, a0f)


def _edge1(f, w, gw, gv, gd, a1f, block_rows=1000):
    E, K = f.shape
    row = lambda i: (i, 0)
    return pl.pallas_call(
        _edge1_body,
        grid=(E // block_rows,),
        in_specs=[pl.BlockSpec((block_rows, K), row),
                  pl.BlockSpec((K, 256), lambda i: (0, 0)),
                  pl.BlockSpec((block_rows, 128), row),
                  pl.BlockSpec((block_rows, 128), row),
                  pl.BlockSpec((block_rows, 128), row),
                  pl.BlockSpec((1, 128), lambda i: (0, 0))],
        out_specs=[pl.BlockSpec((block_rows, 128), row),
                   pl.BlockSpec((block_rows, EXW), row)],
        out_shape=[jax.ShapeDtypeStruct((E, 128), jnp.float32),
                   jax.ShapeDtypeStruct((E, EXW), jnp.float32)],
    )(f, w, gw, gv, gd, a1f)


def _sc_scatter(pv0, pe0, idx0, pv1, pe1, idx1):
    mesh = plsc.VectorSubcoreMesh(core_axis_name="c", subcore_axis_name="s")

    @functools.partial(
        pl.kernel,
        mesh=mesh,
        compiler_params=pltpu.CompilerParams(use_tc_tiling_on_sc=False),
        out_type=[jax.ShapeDtypeStruct((2, N_PAD, 128), jnp.float32),
                  jax.ShapeDtypeStruct((2, N_PAD, EXW), jnp.float32)],
        scratch_types=[
            pltpu.VMEM_SHARED((N_PAD, 128), jnp.float32),
            pltpu.VMEM_SHARED((N_PAD, EXW), jnp.float32),
            pltpu.VMEM((16, 128), jnp.float32),
            pltpu.VMEM((16, EXW), jnp.float32),
            pltpu.VMEM((CH,), jnp.int32),
            pltpu.VMEM((CH,), jnp.int32),
            pltpu.VMEM((CH, 128), jnp.float32),
            pltpu.VMEM((CH, 128), jnp.float32),
            pltpu.VMEM((CH, EXW), jnp.float32),
            pltpu.VMEM((CH, EXW), jnp.float32),
            pltpu.VMEM((8,), jnp.int32),
            pltpu.VMEM((80,), jnp.int32),
            pltpu.SemaphoreType.DMA,
            pltpu.SemaphoreType.DMA,
        ],
    )
    def k(pv0_hbm, pe0_hbm, idx0_hbm, pv1_hbm, pe1_hbm, idx1_hbm,
          outv_hbm, outd_hbm,
          accv, accd, zerov, zerod, idx_a, idx_b, pv_a, pv_b, pe_a, pe_b,
          idx_t0, idx_t1, sem_a, sem_b):
        c = lax.axis_index("c")
        s = lax.axis_index("s")
        wid = c * 16 + s

        for i in range(16):
            for j in range(128 // 16):
                zerov[i, pl.ds(j * 16, 16)] = jnp.zeros((16,), jnp.float32)
            zerod[i, pl.ds(0, EXW)] = jnp.zeros((EXW,), jnp.float32)
        row0 = s * RPT

        def zbody(r, _):
            pltpu.sync_copy(zerov, accv.at[pl.ds(row0 + r * 16, 16)])
            pltpu.sync_copy(zerod, accd.at[pl.ds(row0 + r * 16, 16)])
            return 0
        lax.fori_loop(0, RPT // 16, zbody, 0)
        pltpu.sync_copy(zerov.at[pl.ds(0, 8)],
                        accv.at[pl.ds(row0 + 16 * (RPT // 16), 8)])
        pltpu.sync_copy(zerod.at[pl.ds(0, 8)],
                        accd.at[pl.ds(row0 + 16 * (RPT // 16), 8)])
        plsc.subcore_barrier()

        def seg(pv_hbm, pe_hbm, idx_hbm, base, npairs):
            pltpu.async_copy(idx_hbm.at[pl.ds(base, CH)], idx_a, sem_a)
            pltpu.async_copy(pv_hbm.at[pl.ds(base, CH)], pv_a, sem_a)
            pltpu.async_copy(pe_hbm.at[pl.ds(base, CH)], pe_a, sem_a)

            def body(j, _):
                o1 = base + (2 * j + 1) * CH
                pltpu.async_copy(idx_hbm.at[pl.ds(o1, CH)], idx_b, sem_b)
                pltpu.async_copy(pv_hbm.at[pl.ds(o1, CH)], pv_b, sem_b)
                pltpu.async_copy(pe_hbm.at[pl.ds(o1, CH)], pe_b, sem_b)
                pltpu.make_async_copy(idx_hbm.at[pl.ds(base, CH)], idx_a,
                                      sem_a).wait()
                pltpu.make_async_copy(pv_hbm.at[pl.ds(base, CH)], pv_a,
                                      sem_a).wait()
                pltpu.make_async_copy(pe_hbm.at[pl.ds(base, CH)], pe_a,
                                      sem_a).wait()
                pltpu.sync_copy(pv_a, accv.at[idx_a], add=True)
                pltpu.sync_copy(pe_a, accd.at[idx_a], add=True)

                @pl.when(j < npairs - 1)
                def _():
                    o2 = base + (2 * j + 2) * CH
                    pltpu.async_copy(idx_hbm.at[pl.ds(o2, CH)], idx_a, sem_a)
                    pltpu.async_copy(pv_hbm.at[pl.ds(o2, CH)], pv_a, sem_a)
                    pltpu.async_copy(pe_hbm.at[pl.ds(o2, CH)], pe_a, sem_a)

                pltpu.make_async_copy(idx_hbm.at[pl.ds(o1, CH)], idx_b,
                                      sem_b).wait()
                pltpu.make_async_copy(pv_hbm.at[pl.ds(o1, CH)], pv_b,
                                      sem_b).wait()
                pltpu.make_async_copy(pe_hbm.at[pl.ds(o1, CH)], pe_b,
                                      sem_b).wait()
                pltpu.sync_copy(pv_b, accv.at[idx_b], add=True)
                pltpu.sync_copy(pe_b, accd.at[idx_b], add=True)
                return 0
            lax.fori_loop(0, npairs, body, 0)

        base0 = wid * (E_PP // 32)
        seg(pv0_hbm, pe0_hbm, idx0_hbm, base0, 26)
        t0 = base0 + 52 * CH
        pltpu.sync_copy(idx0_hbm.at[pl.ds(t0, 8)], idx_t0)
        pltpu.sync_copy(pv0_hbm.at[pl.ds(t0, 8)], pv_a.at[pl.ds(0, 8)])
        pltpu.sync_copy(pe0_hbm.at[pl.ds(t0, 8)], pe_a.at[pl.ds(0, 8)])
        pltpu.sync_copy(pv_a.at[pl.ds(0, 8)], accv.at[idx_t0], add=True)
        pltpu.sync_copy(pe_a.at[pl.ds(0, 8)], accd.at[idx_t0], add=True)

        base1 = wid * (E_LP // 32)
        seg(pv1_hbm, pe1_hbm, idx1_hbm, base1, 10)
        t1 = base1 + 20 * CH
        pltpu.sync_copy(idx1_hbm.at[pl.ds(t1, 80)], idx_t1)
        pltpu.sync_copy(pv1_hbm.at[pl.ds(t1, 80)], pv_a.at[pl.ds(0, 80)])
        pltpu.sync_copy(pe1_hbm.at[pl.ds(t1, 80)], pe_a.at[pl.ds(0, 80)])
        pltpu.sync_copy(pv_a.at[pl.ds(0, 80)], accv.at[idx_t1], add=True)
        pltpu.sync_copy(pe_a.at[pl.ds(0, 80)], accd.at[idx_t1], add=True)

        plsc.subcore_barrier()
        pltpu.sync_copy(accv.at[pl.ds(row0, RPT)],
                        outv_hbm.at[c, pl.ds(row0, RPT)])
        pltpu.sync_copy(accd.at[pl.ds(row0, RPT)],
                        outd_hbm.at[c, pl.ds(row0, RPT)])

    return k(pv0, pe0, idx0, pv1, pe1, idx1)


def _head_expand(x):
    r = (jax.lax.broadcasted_iota(jnp.int32, (H, H * DH), 1) // DH
         == jax.lax.broadcasted_iota(jnp.int32, (H, H * DH), 0)
         ).astype(jnp.float32)
    return jnp.dot(x, r, preferred_element_type=jnp.float32)


def _tail_body(v0_ref, v1_ref, d0_ref, d1_ref, pv0_ref, pv1_ref, prot_ref,
               pvec_ref, wh0_ref, bh0_ref, wh1_ref, bh1_ref, wu0_ref,
               bu0_ref, wu1_ref, bu1_ref, wg_ref, bg_ref,
               os_ref, ov_ref):
    t = v0_ref[0] + v1_ref[0]
    d = d0_ref[0] + d1_ref[0]
    den0 = d[:, :H]
    den1 = d[:, H:2 * H]
    numf = (t + _head_expand(den0) * pv0_ref[...]
            + _head_expand(den1) * pv1_ref[...])
    agg = numf / (_head_expand(den0 + den1) + 1e-9)
    h = jax.nn.gelu(jnp.dot(agg, wh0_ref[...],
                            preferred_element_type=jnp.float32) + bh0_ref[...])
    h = jax.nn.gelu(jnp.dot(h, wh1_ref[...],
                            preferred_element_type=jnp.float32) + bh1_ref[...])
    prot = prot_ref[...]
    u = jax.nn.gelu(jnp.dot(jnp.concatenate([prot, h], axis=-1), wu0_ref[...],
                            preferred_element_type=jnp.float32) + bu0_ref[...])
    u = jnp.dot(u, wu1_ref[...], preferred_element_type=jnp.float32) + bu1_ref[...]
    ns = prot + u
    g = jax.nn.sigmoid(jnp.dot(ns, wg_ref[...],
                               preferred_element_type=jnp.float32) + bg_ref[...])
    r3 = (jax.lax.broadcasted_iota(jnp.int32, (H, 3 * H), 1) // 3
          == jax.lax.broadcasted_iota(jnp.int32, (H, 3 * H), 0)
          ).astype(jnp.float32)
    gexp = jnp.dot(g, r3, preferred_element_type=jnp.float32)
    os_ref[...] = ns
    ov_ref[...] = pvec_ref[...] * gexp


def _tail(pv, pd, pv0, pv1, prot, pvec12,
          Wh0, bh0, Wh1, bh1, Wu0, bu0, Wu1, bu1, Wg, bg, block_rows=1000):
    N = prot.shape[0]
    D = prot.shape[1]
    grid = N // block_rows
    row = lambda i: (i, 0)
    full = lambda shape: pl.BlockSpec(shape, lambda i: (0, 0))
    return pl.pallas_call(
        _tail_body,
        grid=(grid,),
        in_specs=[
            pl.BlockSpec((1, block_rows, 128), lambda i: (0, i, 0)),
            pl.BlockSpec((1, block_rows, 128), lambda i: (1, i, 0)),
            pl.BlockSpec((1, block_rows, EXW), lambda i: (0, i, 0)),
            pl.BlockSpec((1, block_rows, EXW), lambda i: (1, i, 0)),
            pl.BlockSpec((block_rows, D), row),
            pl.BlockSpec((block_rows, D), row),
            pl.BlockSpec((block_rows, D), row),
            pl.BlockSpec((block_rows, 3 * H), row),
            full((D, D)), full((1, D)),
            full((D, D)), full((1, D)),
            full((2 * D, D)), full((1, D)),
            full((D, D)), full((1, D)),
            full((D, H)), full((1, H)),
        ],
        out_specs=[pl.BlockSpec((block_rows, D), row),
                   pl.BlockSpec((block_rows, 3 * H), row)],
        out_shape=[jax.ShapeDtypeStruct((N, D), jnp.float32),
                   jax.ShapeDtypeStruct((N, 3 * H), jnp.float32)],
    )(pv, pv, pd, pd, pv0, pv1, prot, pvec12,
      Wh0, bh0.reshape(1, -1), Wh1, bh1.reshape(1, -1),
      Wu0, bu0.reshape(1, -1), Wu1, bu1.reshape(1, -1),
      Wg, bg.reshape(1, -1))


def kernel(source_node_edge_features_exp, prot_scalars, prot_vectors,
           pr_pr_edge_index, lig_scalars, lig_pr_eattr, lig_pr_edge_index,
           W0, a0, V0, W1, a1, V1, Wh0, bh0, Wh1, bh1, Wu0, bu0, Wu1, bu1,
           Wg, bg):
    feats = source_node_edge_features_exp
    N, D = prot_scalars.shape
    DE = feats.shape[1]

    WV0a = jnp.concatenate([W0[:DE], V0[:DE]], axis=1)
    WV1c = jnp.concatenate([W1[2 * D:], V1[2 * D:]], axis=1)
    PN = jnp.concatenate([W0[DE:], V0[DE:], W1[D:2 * D], V1[D:2 * D]], axis=1)
    PP = _mm(prot_scalars, PN, 1000)
    LN = jnp.concatenate([W1[:D], V1[:D]], axis=1)
    LL = _mm(lig_scalars, LN, 1000)

    PW0, PV0, PW1, PV1 = PP[:, :D], PP[:, D:2*D], PP[:, 2*D:3*D], PP[:, 3*D:]
    LW1, LV1 = LL[:, :D], LL[:, D:]

    dst0 = pr_pr_edge_index[1]
    src1 = lig_pr_edge_index[0]
    dst1 = lig_pr_edge_index[1]

    GW0, GSW, GSV, GD = _sc_gather(PW0, dst0, LL, src1, PW1, dst1)
    pay0v, pay0e = _edge0(feats, WV0a, GW0, a0.reshape(1, H * DH))
    pay1v, pay1e = _edge1(lig_pr_eattr, WV1c, GSW, GSV, GD,
                          a1.reshape(1, H * DH))

    pv, pd = _sc_scatter(pay0v, pay0e, dst0, pay1v, pay1e, dst1)

    new_scalars, nv12 = _tail(
        pv, pd, PV0, PV1, prot_scalars,
        prot_vectors.reshape(N, 3 * H),
        Wh0, bh0, Wh1, bh1, Wu0, bu0, Wu1, bu1, Wg, bg)
    return (new_scalars, nv12.reshape(N, H, 3))


# speedup vs baseline: 1.1063x; 1.1063x over previous
import functools

import jax
import jax.numpy as jnp
from jax import lax
from jax.experimental import pallas as pl
from jax.experimental.pallas import tpu as pltpu
from jax.experimental.pallas import tpu_sc as plsc

H = 4
DH = 32
EXW = 16
CH = 96
E_PP = 160000
E_LP = 64000
N_NODES = 10000
N_PAD = 10112
RPT = 632


def _mm_body(x_ref, w_ref, o_ref):
    o_ref[...] = jnp.dot(x_ref[...], w_ref[...],
                         preferred_element_type=jnp.float32)


def _mm(x, w, block_rows):
    M, K = x.shape
    _, N = w.shape
    assert M % block_rows == 0
    return pl.pallas_call(
        _mm_body,
        grid=(M // block_rows,),
        in_specs=[pl.BlockSpec((block_rows, K), lambda i: (i, 0)),
                  pl.BlockSpec((K, N), lambda i: (0, 0))],
        out_specs=pl.BlockSpec((block_rows, N), lambda i: (i, 0)),
        out_shape=jax.ShapeDtypeStruct((M, N), jnp.float32),
    )(x, w)


GCH = 128


def _sc_gather(pw0, dst0, ll, src1, pw1, dst1):
    mesh = plsc.VectorSubcoreMesh(core_axis_name="c", subcore_axis_name="s")

    @functools.partial(
        pl.kernel,
        mesh=mesh,
        compiler_params=pltpu.CompilerParams(use_tc_tiling_on_sc=False),
        out_type=[jax.ShapeDtypeStruct((E_PP, 128), jnp.float32),
                  jax.ShapeDtypeStruct((E_LP, 128), jnp.float32),
                  jax.ShapeDtypeStruct((E_LP, 128), jnp.float32),
                  jax.ShapeDtypeStruct((E_LP, 128), jnp.float32)],
        scratch_types=[
            pltpu.VMEM((GCH,), jnp.int32),
            pltpu.VMEM((GCH,), jnp.int32),
            pltpu.VMEM((GCH, 128), jnp.float32),
            pltpu.VMEM((GCH, 128), jnp.float32),
            pltpu.VMEM((GCH, 256), jnp.float32),
            pltpu.VMEM((GCH, 256), jnp.float32),
            pltpu.SemaphoreType.DMA,
            pltpu.SemaphoreType.DMA,
        ],
    )
    def k(pw0_hbm, dst0_hbm, ll_hbm, src1_hbm, pw1_hbm, dst1_hbm,
          gw0_hbm, gsw_hbm, gsv_hbm, gd_hbm,
          idx_a, idx_b, n_a, n_b, w_a, w_b, sem_a, sem_b):
        c = lax.axis_index("c")
        s = lax.axis_index("s")
        wid = c * 16 + s

        def gseg(idx_hbm, table_hbm, outs, base, nfull, tail, buf_a, buf_b):
            wide = len(outs) > 1

            def wr(buf, o, n):
                for out_hbm, lo in outs:
                    srcb = (buf.at[pl.ds(0, n), pl.ds(lo, 128)]
                            if (wide or n != GCH) else buf)
                    pltpu.sync_copy(srcb, out_hbm.at[pl.ds(o, n)])

            npairs = nfull // 2
            pltpu.sync_copy(idx_hbm.at[pl.ds(base, GCH)], idx_a)
            pltpu.async_copy(table_hbm.at[idx_a], buf_a, sem_a)

            def body(j, _):
                o0 = base + (2 * j) * GCH
                o1 = base + (2 * j + 1) * GCH
                pltpu.sync_copy(idx_hbm.at[pl.ds(o1, GCH)], idx_b)
                pltpu.async_copy(table_hbm.at[idx_b], buf_b, sem_b)
                pltpu.make_async_copy(table_hbm.at[idx_a], buf_a, sem_a).wait()
                wr(buf_a, o0, GCH)

                @pl.when(j < npairs - 1)
                def _():
                    o2 = base + (2 * j + 2) * GCH
                    pltpu.sync_copy(idx_hbm.at[pl.ds(o2, GCH)], idx_a)
                    pltpu.async_copy(table_hbm.at[idx_a], buf_a, sem_a)

                pltpu.make_async_copy(table_hbm.at[idx_b], buf_b, sem_b).wait()
                wr(buf_b, o1, GCH)
                return 0
            lax.fori_loop(0, npairs, body, 0)

            tails = ([(base + 2 * npairs * GCH, GCH)] if nfull % 2 else [])
            tails.append((base + nfull * GCH, tail))
            for t, n in tails:
                it = idx_a.at[pl.ds(0, n)]
                bt = buf_a.at[pl.ds(0, n)]
                pltpu.sync_copy(idx_hbm.at[pl.ds(t, n)], it)
                pltpu.async_copy(table_hbm.at[it], bt, sem_a)
                pltpu.make_async_copy(table_hbm.at[it], bt, sem_a).wait()
                wr(buf_a, t, n)

        base0 = wid * (E_PP // 32)
        base1 = wid * (E_LP // 32)
        gseg(dst0_hbm, pw0_hbm, [(gw0_hbm, 0)], base0, 39, 8, n_a, n_b)
        gseg(src1_hbm, ll_hbm, [(gsw_hbm, 0), (gsv_hbm, 128)],
             base1, 15, 80, w_a, w_b)
        gseg(dst1_hbm, pw1_hbm, [(gd_hbm, 0)], base1, 15, 80, n_a, n_b)

    return k(pw0, dst0, ll, src1, pw1, dst1)


def _iota2(shape, d0, d1, div):
    return (jax.lax.broadcasted_iota(jnp.int32, shape, d0) // div
            == jax.lax.broadcasted_iota(jnp.int32, shape, d1)
            ).astype(jnp.float32)


def _edge0_body(f_ref, w_ref, g_ref, a_ref, o_ref, e_ref):
    x = jnp.dot(f_ref[...], w_ref[...], preferred_element_type=jnp.float32)
    pre = x[:, :128] + g_ref[...]
    hh = jnp.where(pre >= 0, pre, 0.2 * pre)
    sel = _iota2((H * DH, H), 0, 1, DH)
    ex = jnp.exp(jnp.dot(hh * a_ref[...], sel,
                         preferred_element_type=jnp.float32))
    rexp = _iota2((H, H * DH), 1, 0, DH)
    b = ex.shape[0]
    o_ref[...] = jnp.dot(ex, rexp, preferred_element_type=jnp.float32) * x[:, 128:]
    e_ref[...] = jnp.concatenate(
        [ex, jnp.zeros((b, EXW - H), jnp.float32)], axis=1)


def _edge1_body(f_ref, w_ref, gw_ref, gv_ref, gd_ref, a_ref, o_ref, e_ref):
    x = jnp.dot(f_ref[...], w_ref[...], preferred_element_type=jnp.float32)
    pre = x[:, :128] + gw_ref[...] + gd_ref[...]
    fv = x[:, 128:] + gv_ref[...]
    hh = jnp.where(pre >= 0, pre, 0.2 * pre)
    sel = _iota2((H * DH, H), 0, 1, DH)
    ex = jnp.exp(jnp.dot(hh * a_ref[...], sel,
                         preferred_element_type=jnp.float32))
    rexp = _iota2((H, H * DH), 1, 0, DH)
    b = ex.shape[0]
    o_ref[...] = jnp.dot(ex, rexp, preferred_element_type=jnp.float32) * fv
    e_ref[...] = jnp.concatenate(
        [jnp.zeros((b, H), jnp.float32), ex,
         jnp.zeros((b, EXW - 2 * H), jnp.float32)], axis=1)


def _edge0(f, w, g, a0f, block_rows=2000):
    E, K = f.shape
    row = lambda i: (i, 0)
    return pl.pallas_call(
        _edge0_body,
        grid=(E // block_rows,),
        in_specs=[pl.BlockSpec((block_rows, K), row),
                  pl.BlockSpec((K, 256), lambda i: (0, 0)),
                  pl.BlockSpec((block_rows, 128), row),
                  pl.BlockSpec((1, 128), lambda i: (0, 0))],
        out_specs=[pl.BlockSpec((block_rows, 128), row),
                   pl.BlockSpec((block_rows, EXW), row)],
        out_shape=[jax.ShapeDtypeStruct((E, 128), jnp.float32),
                   jax.ShapeDtypeStruct((E, EXW), jnp.float32)],
    )(f, w, g, a0f)


def _edge1(f, w, gw, gv, gd, a1f, block_rows=2000):
    E, K = f.shape
    row = lambda i: (i, 0)
    return pl.pallas_call(
        _edge1_body,
        grid=(E // block_rows,),
        in_specs=[pl.BlockSpec((block_rows, K), row),
                  pl.BlockSpec((K, 256), lambda i: (0, 0)),
                  pl.BlockSpec((block_rows, 128), row),
                  pl.BlockSpec((block_rows, 128), row),
                  pl.BlockSpec((block_rows, 128), row),
                  pl.BlockSpec((1, 128), lambda i: (0, 0))],
        out_specs=[pl.BlockSpec((block_rows, 128), row),
                   pl.BlockSpec((block_rows, EXW), row)],
        out_shape=[jax.ShapeDtypeStruct((E, 128), jnp.float32),
                   jax.ShapeDtypeStruct((E, EXW), jnp.float32)],
    )(f, w, gw, gv, gd, a1f)


def _sc_scatter(pv0, pe0, idx0, pv1, pe1, idx1):
    mesh = plsc.VectorSubcoreMesh(core_axis_name="c", subcore_axis_name="s")

    @functools.partial(
        pl.kernel,
        mesh=mesh,
        compiler_params=pltpu.CompilerParams(use_tc_tiling_on_sc=False),
        out_type=[jax.ShapeDtypeStruct((2, N_PAD, 128), jnp.float32),
                  jax.ShapeDtypeStruct((2, N_PAD, EXW), jnp.float32)],
        scratch_types=[
            pltpu.VMEM_SHARED((N_PAD, 128), jnp.float32),
            pltpu.VMEM_SHARED((N_PAD, EXW), jnp.float32),
            pltpu.VMEM((16, 128), jnp.float32),
            pltpu.VMEM((16, EXW), jnp.float32),
            pltpu.VMEM((CH,), jnp.int32),
            pltpu.VMEM((CH,), jnp.int32),
            pltpu.VMEM((CH, 128), jnp.float32),
            pltpu.VMEM((CH, 128), jnp.float32),
            pltpu.VMEM((CH, EXW), jnp.float32),
            pltpu.VMEM((CH, EXW), jnp.float32),
            pltpu.VMEM((8,), jnp.int32),
            pltpu.VMEM((80,), jnp.int32),
            pltpu.SemaphoreType.DMA,
            pltpu.SemaphoreType.DMA,
        ],
    )
    def k(pv0_hbm, pe0_hbm, idx0_hbm, pv1_hbm, pe1_hbm, idx1_hbm,
          outv_hbm, outd_hbm,
          accv, accd, zerov, zerod, idx_a, idx_b, pv_a, pv_b, pe_a, pe_b,
          idx_t0, idx_t1, sem_a, sem_b):
        c = lax.axis_index("c")
        s = lax.axis_index("s")
        wid = c * 16 + s

        for i in range(16):
            for j in range(128 // 16):
                zerov[i, pl.ds(j * 16, 16)] = jnp.zeros((16,), jnp.float32)
            zerod[i, pl.ds(0, EXW)] = jnp.zeros((EXW,), jnp.float32)
        row0 = s * RPT

        def zbody(r, _):
            pltpu.sync_copy(zerov, accv.at[pl.ds(row0 + r * 16, 16)])
            pltpu.sync_copy(zerod, accd.at[pl.ds(row0 + r * 16, 16)])
            return 0
        lax.fori_loop(0, RPT // 16, zbody, 0)
        pltpu.sync_copy(zerov.at[pl.ds(0, 8)],
                        accv.at[pl.ds(row0 + 16 * (RPT // 16), 8)])
        pltpu.sync_copy(zerod.at[pl.ds(0, 8)],
                        accd.at[pl.ds(row0 + 16 * (RPT // 16), 8)])
        plsc.subcore_barrier()

        def seg(pv_hbm, pe_hbm, idx_hbm, base, npairs):
            pltpu.async_copy(idx_hbm.at[pl.ds(base, CH)], idx_a, sem_a)
            pltpu.async_copy(pv_hbm.at[pl.ds(base, CH)], pv_a, sem_a)
            pltpu.async_copy(pe_hbm.at[pl.ds(base, CH)], pe_a, sem_a)

            def body(j, _):
                o1 = base + (2 * j + 1) * CH
                pltpu.async_copy(idx_hbm.at[pl.ds(o1, CH)], idx_b, sem_b)
                pltpu.async_copy(pv_hbm.at[pl.ds(o1, CH)], pv_b, sem_b)
                pltpu.async_copy(pe_hbm.at[pl.ds(o1, CH)], pe_b, sem_b)
                pltpu.make_async_copy(idx_hbm.at[pl.ds(base, CH)], idx_a,
                                      sem_a).wait()
                pltpu.make_async_copy(pv_hbm.at[pl.ds(base, CH)], pv_a,
                                      sem_a).wait()
                pltpu.make_async_copy(pe_hbm.at[pl.ds(base, CH)], pe_a,
                                      sem_a).wait()
                pltpu.sync_copy(pv_a, accv.at[idx_a], add=True)
                pltpu.sync_copy(pe_a, accd.at[idx_a], add=True)

                @pl.when(j < npairs - 1)
                def _():
                    o2 = base + (2 * j + 2) * CH
                    pltpu.async_copy(idx_hbm.at[pl.ds(o2, CH)], idx_a, sem_a)
                    pltpu.async_copy(pv_hbm.at[pl.ds(o2, CH)], pv_a, sem_a)
                    pltpu.async_copy(pe_hbm.at[pl.ds(o2, CH)], pe_a, sem_a)

                pltpu.make_async_copy(idx_hbm.at[pl.ds(o1, CH)], idx_b,
                                      sem_b).wait()
                pltpu.make_async_copy(pv_hbm.at[pl.ds(o1, CH)], pv_b,
                                      sem_b).wait()
                pltpu.make_async_copy(pe_hbm.at[pl.ds(o1, CH)], pe_b,
                                      sem_b).wait()
                pltpu.sync_copy(pv_b, accv.at[idx_b], add=True)
                pltpu.sync_copy(pe_b, accd.at[idx_b], add=True)
                return 0
            lax.fori_loop(0, npairs, body, 0)

        base0 = wid * (E_PP // 32)
        seg(pv0_hbm, pe0_hbm, idx0_hbm, base0, 26)
        t0 = base0 + 52 * CH
        pltpu.sync_copy(idx0_hbm.at[pl.ds(t0, 8)], idx_t0)
        pltpu.sync_copy(pv0_hbm.at[pl.ds(t0, 8)], pv_a.at[pl.ds(0, 8)])
        pltpu.sync_copy(pe0_hbm.at[pl.ds(t0, 8)], pe_a.at[pl.ds(0, 8)])
        pltpu.sync_copy(pv_a.at[pl.ds(0, 8)], accv.at[idx_t0], add=True)
        pltpu.sync_copy(pe_a.at[pl.ds(0, 8)], accd.at[idx_t0], add=True)

        base1 = wid * (E_LP // 32)
        seg(pv1_hbm, pe1_hbm, idx1_hbm, base1, 10)
        t1 = base1 + 20 * CH
        pltpu.sync_copy(idx1_hbm.at[pl.ds(t1, 80)], idx_t1)
        pltpu.sync_copy(pv1_hbm.at[pl.ds(t1, 80)], pv_a.at[pl.ds(0, 80)])
        pltpu.sync_copy(pe1_hbm.at[pl.ds(t1, 80)], pe_a.at[pl.ds(0, 80)])
        pltpu.sync_copy(pv_a.at[pl.ds(0, 80)], accv.at[idx_t1], add=True)
        pltpu.sync_copy(pe_a.at[pl.ds(0, 80)], accd.at[idx_t1], add=True)

        plsc.subcore_barrier()
        pltpu.sync_copy(accv.at[pl.ds(row0, RPT)],
                        outv_hbm.at[c, pl.ds(row0, RPT)])
        pltpu.sync_copy(accd.at[pl.ds(row0, RPT)],
                        outd_hbm.at[c, pl.ds(row0, RPT)])

    return k(pv0, pe0, idx0, pv1, pe1, idx1)


def _head_expand(x):
    r = (jax.lax.broadcasted_iota(jnp.int32, (H, H * DH), 1) // DH
         == jax.lax.broadcasted_iota(jnp.int32, (H, H * DH), 0)
         ).astype(jnp.float32)
    return jnp.dot(x, r, preferred_element_type=jnp.float32)


def _tail_body(v0_ref, v1_ref, d0_ref, d1_ref, pv0_ref, pv1_ref, prot_ref,
               pvec_ref, wh0_ref, bh0_ref, wh1_ref, bh1_ref, wu0_ref,
               bu0_ref, wu1_ref, bu1_ref, wg_ref, bg_ref,
               os_ref, ov_ref):
    t = v0_ref[0] + v1_ref[0]
    d = d0_ref[0] + d1_ref[0]
    den0 = d[:, :H]
    den1 = d[:, H:2 * H]
    numf = (t + _head_expand(den0) * pv0_ref[...]
            + _head_expand(den1) * pv1_ref[...])
    agg = numf / (_head_expand(den0 + den1) + 1e-9)
    h = jax.nn.gelu(jnp.dot(agg, wh0_ref[...],
                            preferred_element_type=jnp.float32) + bh0_ref[...])
    h = jax.nn.gelu(jnp.dot(h, wh1_ref[...],
                            preferred_element_type=jnp.float32) + bh1_ref[...])
    prot = prot_ref[...]
    u = jax.nn.gelu(jnp.dot(jnp.concatenate([prot, h], axis=-1), wu0_ref[...],
                            preferred_element_type=jnp.float32) + bu0_ref[...])
    u = jnp.dot(u, wu1_ref[...], preferred_element_type=jnp.float32) + bu1_ref[...]
    ns = prot + u
    g = jax.nn.sigmoid(jnp.dot(ns, wg_ref[...],
                               preferred_element_type=jnp.float32) + bg_ref[...])
    r3 = (jax.lax.broadcasted_iota(jnp.int32, (H, 3 * H), 1) // 3
          == jax.lax.broadcasted_iota(jnp.int32, (H, 3 * H), 0)
          ).astype(jnp.float32)
    gexp = jnp.dot(g, r3, preferred_element_type=jnp.float32)
    os_ref[...] = ns
    ov_ref[...] = pvec_ref[...] * gexp


def _tail(pv, pd, pv0, pv1, prot, pvec12,
          Wh0, bh0, Wh1, bh1, Wu0, bu0, Wu1, bu1, Wg, bg, block_rows=1000):
    N = prot.shape[0]
    D = prot.shape[1]
    grid = N // block_rows
    row = lambda i: (i, 0)
    full = lambda shape: pl.BlockSpec(shape, lambda i: (0, 0))
    return pl.pallas_call(
        _tail_body,
        grid=(grid,),
        in_specs=[
            pl.BlockSpec((1, block_rows, 128), lambda i: (0, i, 0)),
            pl.BlockSpec((1, block_rows, 128), lambda i: (1, i, 0)),
            pl.BlockSpec((1, block_rows, EXW), lambda i: (0, i, 0)),
            pl.BlockSpec((1, block_rows, EXW), lambda i: (1, i, 0)),
            pl.BlockSpec((block_rows, D), row),
            pl.BlockSpec((block_rows, D), row),
            pl.BlockSpec((block_rows, D), row),
            pl.BlockSpec((block_rows, 3 * H), row),
            full((D, D)), full((1, D)),
            full((D, D)), full((1, D)),
            full((2 * D, D)), full((1, D)),
            full((D, D)), full((1, D)),
            full((D, H)), full((1, H)),
        ],
        out_specs=[pl.BlockSpec((block_rows, D), row),
                   pl.BlockSpec((block_rows, 3 * H), row)],
        out_shape=[jax.ShapeDtypeStruct((N, D), jnp.float32),
                   jax.ShapeDtypeStruct((N, 3 * H), jnp.float32)],
    )(pv, pv, pd, pd, pv0, pv1, prot, pvec12,
      Wh0, bh0.reshape(1, -1), Wh1, bh1.reshape(1, -1),
      Wu0, bu0.reshape(1, -1), Wu1, bu1.reshape(1, -1),
      Wg, bg.reshape(1, -1))


def kernel(source_node_edge_features_exp, prot_scalars, prot_vectors,
           pr_pr_edge_index, lig_scalars, lig_pr_eattr, lig_pr_edge_index,
           W0, a0, V0, W1, a1, V1, Wh0, bh0, Wh1, bh1, Wu0, bu0, Wu1, bu1,
           Wg, bg):
    feats = source_node_edge_features_exp
    N, D = prot_scalars.shape
    DE = feats.shape[1]

    WV0a = jnp.concatenate([W0[:DE], V0[:DE]], axis=1)
    WV1c = jnp.concatenate([W1[2 * D:], V1[2 * D:]], axis=1)
    PN = jnp.concatenate([W0[DE:], V0[DE:], W1[D:2 * D], V1[D:2 * D]], axis=1)
    PP = _mm(prot_scalars, PN, 1000)
    LN = jnp.concatenate([W1[:D], V1[:D]], axis=1)
    LL = _mm(lig_scalars, LN, 1000)

    PW0, PV0, PW1, PV1 = PP[:, :D], PP[:, D:2*D], PP[:, 2*D:3*D], PP[:, 3*D:]
    LW1, LV1 = LL[:, :D], LL[:, D:]

    dst0 = pr_pr_edge_index[1]
    src1 = lig_pr_edge_index[0]
    dst1 = lig_pr_edge_index[1]

    GW0, GSW, GSV, GD = _sc_gather(PW0, dst0, LL, src1, PW1, dst1)
    pay0v, pay0e = _edge0(feats, WV0a, GW0, a0.reshape(1, H * DH))
    pay1v, pay1e = _edge1(lig_pr_eattr, WV1c, GSW, GSV, GD,
                          a1.reshape(1, H * DH))

    pv, pd = _sc_scatter(pay0v, pay0e, dst0, pay1v, pay1e, dst1)

    new_scalars, nv12 = _tail(
        pv, pd, PV0, PV1, prot_scalars,
        prot_vectors.reshape(N, 3 * H),
        Wh0, bh0, Wh1, bh1, Wu0, bu0, Wu1, bu1, Wg, bg)
    return (new_scalars, nv12.reshape(N, H, 3))


# speedup vs baseline: 1.1476x; 1.0373x over previous
import functools

import jax
import jax.numpy as jnp
from jax import lax
from jax.experimental import pallas as pl
from jax.experimental.pallas import tpu as pltpu
from jax.experimental.pallas import tpu_sc as plsc

H = 4
DH = 32
EXW = 16
CH = 96
E_PP = 160000
E_LP = 64000
N_NODES = 10000
N_PAD = 10112
RPT = 632


def _mm_body(x_ref, w_ref, o_ref):
    o_ref[...] = jnp.dot(x_ref[...], w_ref[...],
                         preferred_element_type=jnp.float32)


def _mm(x, w, block_rows):
    M, K = x.shape
    _, N = w.shape
    assert M % block_rows == 0
    return pl.pallas_call(
        _mm_body,
        grid=(M // block_rows,),
        in_specs=[pl.BlockSpec((block_rows, K), lambda i: (i, 0)),
                  pl.BlockSpec((K, N), lambda i: (0, 0))],
        out_specs=pl.BlockSpec((block_rows, N), lambda i: (i, 0)),
        out_shape=jax.ShapeDtypeStruct((M, N), jnp.float32),
    )(x, w)


GCH = 128


def _sc_gather(pw0, dst0, ll, src1, pw1, dst1):
    mesh = plsc.VectorSubcoreMesh(core_axis_name="c", subcore_axis_name="s")

    @functools.partial(
        pl.kernel,
        mesh=mesh,
        compiler_params=pltpu.CompilerParams(use_tc_tiling_on_sc=False),
        out_type=[jax.ShapeDtypeStruct((E_PP, 128), jnp.float32),
                  jax.ShapeDtypeStruct((E_LP, 128), jnp.float32),
                  jax.ShapeDtypeStruct((E_LP, 128), jnp.float32),
                  jax.ShapeDtypeStruct((E_LP, 128), jnp.float32)],
        scratch_types=[
            pltpu.VMEM((GCH,), jnp.int32),
            pltpu.VMEM((GCH,), jnp.int32),
            pltpu.VMEM((GCH, 128), jnp.float32),
            pltpu.VMEM((GCH, 128), jnp.float32),
            pltpu.VMEM((GCH, 256), jnp.float32),
            pltpu.VMEM((GCH, 256), jnp.float32),
            pltpu.SemaphoreType.DMA,
            pltpu.SemaphoreType.DMA,
        ],
    )
    def k(pw0_hbm, dst0_hbm, ll_hbm, src1_hbm, pw1_hbm, dst1_hbm,
          gw0_hbm, gsw_hbm, gsv_hbm, gd_hbm,
          idx_a, idx_b, n_a, n_b, w_a, w_b, sem_a, sem_b):
        c = lax.axis_index("c")
        s = lax.axis_index("s")
        wid = c * 16 + s

        def gseg(idx_hbm, table_hbm, outs, base, nfull, tail, buf_a, buf_b):
            wide = len(outs) > 1

            def wr(buf, o, n):
                for out_hbm, lo in outs:
                    srcb = (buf.at[pl.ds(0, n), pl.ds(lo, 128)]
                            if (wide or n != GCH) else buf)
                    pltpu.sync_copy(srcb, out_hbm.at[pl.ds(o, n)])

            npairs = nfull // 2
            pltpu.sync_copy(idx_hbm.at[pl.ds(base, GCH)], idx_a)
            pltpu.async_copy(table_hbm.at[idx_a], buf_a, sem_a)

            def body(j, _):
                o0 = base + (2 * j) * GCH
                o1 = base + (2 * j + 1) * GCH
                pltpu.sync_copy(idx_hbm.at[pl.ds(o1, GCH)], idx_b)
                pltpu.async_copy(table_hbm.at[idx_b], buf_b, sem_b)
                pltpu.make_async_copy(table_hbm.at[idx_a], buf_a, sem_a).wait()
                wr(buf_a, o0, GCH)

                @pl.when(j < npairs - 1)
                def _():
                    o2 = base + (2 * j + 2) * GCH
                    pltpu.sync_copy(idx_hbm.at[pl.ds(o2, GCH)], idx_a)
                    pltpu.async_copy(table_hbm.at[idx_a], buf_a, sem_a)

                pltpu.make_async_copy(table_hbm.at[idx_b], buf_b, sem_b).wait()
                wr(buf_b, o1, GCH)
                return 0
            lax.fori_loop(0, npairs, body, 0)

            tails = ([(base + 2 * npairs * GCH, GCH)] if nfull % 2 else [])
            tails.append((base + nfull * GCH, tail))
            for t, n in tails:
                it = idx_a.at[pl.ds(0, n)]
                bt = buf_a.at[pl.ds(0, n)]
                pltpu.sync_copy(idx_hbm.at[pl.ds(t, n)], it)
                pltpu.async_copy(table_hbm.at[it], bt, sem_a)
                pltpu.make_async_copy(table_hbm.at[it], bt, sem_a).wait()
                wr(buf_a, t, n)

        base0 = wid * (E_PP // 32)
        base1 = wid * (E_LP // 32)
        gseg(dst0_hbm, pw0_hbm, [(gw0_hbm, 0)], base0, 39, 8, n_a, n_b)
        gseg(src1_hbm, ll_hbm, [(gsw_hbm, 0), (gsv_hbm, 128)],
             base1, 15, 80, w_a, w_b)
        gseg(dst1_hbm, pw1_hbm, [(gd_hbm, 0)], base1, 15, 80, n_a, n_b)

    return k(pw0, dst0, ll, src1, pw1, dst1)


def _iota2(shape, d0, d1, div):
    return (jax.lax.broadcasted_iota(jnp.int32, shape, d0) // div
            == jax.lax.broadcasted_iota(jnp.int32, shape, d1)
            ).astype(jnp.float32)


def _edge0_body(f_ref, w_ref, g_ref, a_ref, o_ref, e_ref):
    x = jnp.dot(f_ref[...], w_ref[...], preferred_element_type=jnp.float32)
    pre = x[:, :128] + g_ref[...]
    hh = jnp.where(pre >= 0, pre, 0.2 * pre)
    sel = _iota2((H * DH, H), 0, 1, DH)
    ex = jnp.exp(jnp.dot(hh * a_ref[...], sel,
                         preferred_element_type=jnp.float32))
    rexp = _iota2((H, H * DH), 1, 0, DH)
    b = ex.shape[0]
    o_ref[...] = jnp.dot(ex, rexp, preferred_element_type=jnp.float32) * x[:, 128:]
    e_ref[...] = jnp.concatenate(
        [ex, jnp.zeros((b, EXW - H), jnp.float32)], axis=1)


def _edge1_body(f_ref, w_ref, gw_ref, gv_ref, gd_ref, a_ref, o_ref, e_ref):
    x = jnp.dot(f_ref[...], w_ref[...], preferred_element_type=jnp.float32)
    pre = x[:, :128] + gw_ref[...] + gd_ref[...]
    fv = x[:, 128:] + gv_ref[...]
    hh = jnp.where(pre >= 0, pre, 0.2 * pre)
    sel = _iota2((H * DH, H), 0, 1, DH)
    ex = jnp.exp(jnp.dot(hh * a_ref[...], sel,
                         preferred_element_type=jnp.float32))
    rexp = _iota2((H, H * DH), 1, 0, DH)
    b = ex.shape[0]
    o_ref[...] = jnp.dot(ex, rexp, preferred_element_type=jnp.float32) * fv
    e_ref[...] = jnp.concatenate(
        [jnp.zeros((b, H), jnp.float32), ex,
         jnp.zeros((b, EXW - 2 * H), jnp.float32)], axis=1)


def _edge0(f, w, g, a0f, block_rows=4000):
    E, K = f.shape
    row = lambda i: (i, 0)
    return pl.pallas_call(
        _edge0_body,
        grid=(E // block_rows,),
        in_specs=[pl.BlockSpec((block_rows, K), row),
                  pl.BlockSpec((K, 256), lambda i: (0, 0)),
                  pl.BlockSpec((block_rows, 128), row),
                  pl.BlockSpec((1, 128), lambda i: (0, 0))],
        out_specs=[pl.BlockSpec((block_rows, 128), row),
                   pl.BlockSpec((block_rows, EXW), row)],
        out_shape=[jax.ShapeDtypeStruct((E, 128), jnp.float32),
                   jax.ShapeDtypeStruct((E, EXW), jnp.float32)],
    )(f, w, g, a0f)


def _edge1(f, w, gw, gv, gd, a1f, block_rows=4000):
    E, K = f.shape
    row = lambda i: (i, 0)
    return pl.pallas_call(
        _edge1_body,
        grid=(E // block_rows,),
        in_specs=[pl.BlockSpec((block_rows, K), row),
                  pl.BlockSpec((K, 256), lambda i: (0, 0)),
                  pl.BlockSpec((block_rows, 128), row),
                  pl.BlockSpec((block_rows, 128), row),
                  pl.BlockSpec((block_rows, 128), row),
                  pl.BlockSpec((1, 128), lambda i: (0, 0))],
        out_specs=[pl.BlockSpec((block_rows, 128), row),
                   pl.BlockSpec((block_rows, EXW), row)],
        out_shape=[jax.ShapeDtypeStruct((E, 128), jnp.float32),
                   jax.ShapeDtypeStruct((E, EXW), jnp.float32)],
    )(f, w, gw, gv, gd, a1f)


def _sc_scatter(pv0, pe0, idx0, pv1, pe1, idx1):
    mesh = plsc.VectorSubcoreMesh(core_axis_name="c", subcore_axis_name="s")

    @functools.partial(
        pl.kernel,
        mesh=mesh,
        compiler_params=pltpu.CompilerParams(use_tc_tiling_on_sc=False),
        out_type=[jax.ShapeDtypeStruct((2, N_PAD, 128), jnp.float32),
                  jax.ShapeDtypeStruct((2, N_PAD, EXW), jnp.float32)],
        scratch_types=[
            pltpu.VMEM_SHARED((N_PAD, 128), jnp.float32),
            pltpu.VMEM_SHARED((N_PAD, EXW), jnp.float32),
            pltpu.VMEM((16, 128), jnp.float32),
            pltpu.VMEM((16, EXW), jnp.float32),
            pltpu.VMEM((CH,), jnp.int32),
            pltpu.VMEM((CH,), jnp.int32),
            pltpu.VMEM((CH, 128), jnp.float32),
            pltpu.VMEM((CH, 128), jnp.float32),
            pltpu.VMEM((CH, EXW), jnp.float32),
            pltpu.VMEM((CH, EXW), jnp.float32),
            pltpu.VMEM((8,), jnp.int32),
            pltpu.VMEM((80,), jnp.int32),
            pltpu.SemaphoreType.DMA,
            pltpu.SemaphoreType.DMA,
        ],
    )
    def k(pv0_hbm, pe0_hbm, idx0_hbm, pv1_hbm, pe1_hbm, idx1_hbm,
          outv_hbm, outd_hbm,
          accv, accd, zerov, zerod, idx_a, idx_b, pv_a, pv_b, pe_a, pe_b,
          idx_t0, idx_t1, sem_a, sem_b):
        c = lax.axis_index("c")
        s = lax.axis_index("s")
        wid = c * 16 + s

        for i in range(16):
            for j in range(128 // 16):
                zerov[i, pl.ds(j * 16, 16)] = jnp.zeros((16,), jnp.float32)
            zerod[i, pl.ds(0, EXW)] = jnp.zeros((EXW,), jnp.float32)
        row0 = s * RPT

        def zbody(r, _):
            pltpu.sync_copy(zerov, accv.at[pl.ds(row0 + r * 16, 16)])
            pltpu.sync_copy(zerod, accd.at[pl.ds(row0 + r * 16, 16)])
            return 0
        lax.fori_loop(0, RPT // 16, zbody, 0)
        pltpu.sync_copy(zerov.at[pl.ds(0, 8)],
                        accv.at[pl.ds(row0 + 16 * (RPT // 16), 8)])
        pltpu.sync_copy(zerod.at[pl.ds(0, 8)],
                        accd.at[pl.ds(row0 + 16 * (RPT // 16), 8)])
        plsc.subcore_barrier()

        def seg(pv_hbm, pe_hbm, idx_hbm, base, npairs):
            pltpu.async_copy(idx_hbm.at[pl.ds(base, CH)], idx_a, sem_a)
            pltpu.async_copy(pv_hbm.at[pl.ds(base, CH)], pv_a, sem_a)
            pltpu.async_copy(pe_hbm.at[pl.ds(base, CH)], pe_a, sem_a)

            def body(j, _):
                o1 = base + (2 * j + 1) * CH
                pltpu.async_copy(idx_hbm.at[pl.ds(o1, CH)], idx_b, sem_b)
                pltpu.async_copy(pv_hbm.at[pl.ds(o1, CH)], pv_b, sem_b)
                pltpu.async_copy(pe_hbm.at[pl.ds(o1, CH)], pe_b, sem_b)
                pltpu.make_async_copy(idx_hbm.at[pl.ds(base, CH)], idx_a,
                                      sem_a).wait()
                pltpu.make_async_copy(pv_hbm.at[pl.ds(base, CH)], pv_a,
                                      sem_a).wait()
                pltpu.make_async_copy(pe_hbm.at[pl.ds(base, CH)], pe_a,
                                      sem_a).wait()
                pltpu.sync_copy(pv_a, accv.at[idx_a], add=True)
                pltpu.sync_copy(pe_a, accd.at[idx_a], add=True)

                @pl.when(j < npairs - 1)
                def _():
                    o2 = base + (2 * j + 2) * CH
                    pltpu.async_copy(idx_hbm.at[pl.ds(o2, CH)], idx_a, sem_a)
                    pltpu.async_copy(pv_hbm.at[pl.ds(o2, CH)], pv_a, sem_a)
                    pltpu.async_copy(pe_hbm.at[pl.ds(o2, CH)], pe_a, sem_a)

                pltpu.make_async_copy(idx_hbm.at[pl.ds(o1, CH)], idx_b,
                                      sem_b).wait()
                pltpu.make_async_copy(pv_hbm.at[pl.ds(o1, CH)], pv_b,
                                      sem_b).wait()
                pltpu.make_async_copy(pe_hbm.at[pl.ds(o1, CH)], pe_b,
                                      sem_b).wait()
                pltpu.sync_copy(pv_b, accv.at[idx_b], add=True)
                pltpu.sync_copy(pe_b, accd.at[idx_b], add=True)
                return 0
            lax.fori_loop(0, npairs, body, 0)

        base0 = wid * (E_PP // 32)
        seg(pv0_hbm, pe0_hbm, idx0_hbm, base0, 26)
        t0 = base0 + 52 * CH
        pltpu.sync_copy(idx0_hbm.at[pl.ds(t0, 8)], idx_t0)
        pltpu.sync_copy(pv0_hbm.at[pl.ds(t0, 8)], pv_a.at[pl.ds(0, 8)])
        pltpu.sync_copy(pe0_hbm.at[pl.ds(t0, 8)], pe_a.at[pl.ds(0, 8)])
        pltpu.sync_copy(pv_a.at[pl.ds(0, 8)], accv.at[idx_t0], add=True)
        pltpu.sync_copy(pe_a.at[pl.ds(0, 8)], accd.at[idx_t0], add=True)

        base1 = wid * (E_LP // 32)
        seg(pv1_hbm, pe1_hbm, idx1_hbm, base1, 10)
        t1 = base1 + 20 * CH
        pltpu.sync_copy(idx1_hbm.at[pl.ds(t1, 80)], idx_t1)
        pltpu.sync_copy(pv1_hbm.at[pl.ds(t1, 80)], pv_a.at[pl.ds(0, 80)])
        pltpu.sync_copy(pe1_hbm.at[pl.ds(t1, 80)], pe_a.at[pl.ds(0, 80)])
        pltpu.sync_copy(pv_a.at[pl.ds(0, 80)], accv.at[idx_t1], add=True)
        pltpu.sync_copy(pe_a.at[pl.ds(0, 80)], accd.at[idx_t1], add=True)

        plsc.subcore_barrier()
        pltpu.sync_copy(accv.at[pl.ds(row0, RPT)],
                        outv_hbm.at[c, pl.ds(row0, RPT)])
        pltpu.sync_copy(accd.at[pl.ds(row0, RPT)],
                        outd_hbm.at[c, pl.ds(row0, RPT)])

    return k(pv0, pe0, idx0, pv1, pe1, idx1)


def _head_expand(x):
    r = (jax.lax.broadcasted_iota(jnp.int32, (H, H * DH), 1) // DH
         == jax.lax.broadcasted_iota(jnp.int32, (H, H * DH), 0)
         ).astype(jnp.float32)
    return jnp.dot(x, r, preferred_element_type=jnp.float32)


def _tail_body(v0_ref, v1_ref, d0_ref, d1_ref, pv0_ref, pv1_ref, prot_ref,
               pvec_ref, wh0_ref, bh0_ref, wh1_ref, bh1_ref, wu0_ref,
               bu0_ref, wu1_ref, bu1_ref, wg_ref, bg_ref,
               os_ref, ov_ref):
    t = v0_ref[0] + v1_ref[0]
    d = d0_ref[0] + d1_ref[0]
    den0 = d[:, :H]
    den1 = d[:, H:2 * H]
    numf = (t + _head_expand(den0) * pv0_ref[...]
            + _head_expand(den1) * pv1_ref[...])
    agg = numf / (_head_expand(den0 + den1) + 1e-9)
    h = jax.nn.gelu(jnp.dot(agg, wh0_ref[...],
                            preferred_element_type=jnp.float32) + bh0_ref[...])
    h = jax.nn.gelu(jnp.dot(h, wh1_ref[...],
                            preferred_element_type=jnp.float32) + bh1_ref[...])
    prot = prot_ref[...]
    u = jax.nn.gelu(jnp.dot(jnp.concatenate([prot, h], axis=-1), wu0_ref[...],
                            preferred_element_type=jnp.float32) + bu0_ref[...])
    u = jnp.dot(u, wu1_ref[...], preferred_element_type=jnp.float32) + bu1_ref[...]
    ns = prot + u
    g = jax.nn.sigmoid(jnp.dot(ns, wg_ref[...],
                               preferred_element_type=jnp.float32) + bg_ref[...])
    r3 = (jax.lax.broadcasted_iota(jnp.int32, (H, 3 * H), 1) // 3
          == jax.lax.broadcasted_iota(jnp.int32, (H, 3 * H), 0)
          ).astype(jnp.float32)
    gexp = jnp.dot(g, r3, preferred_element_type=jnp.float32)
    os_ref[...] = ns
    ov_ref[...] = pvec_ref[...] * gexp


def _tail(pv, pd, pv0, pv1, prot, pvec12,
          Wh0, bh0, Wh1, bh1, Wu0, bu0, Wu1, bu1, Wg, bg, block_rows=2000):
    N = prot.shape[0]
    D = prot.shape[1]
    grid = N // block_rows
    row = lambda i: (i, 0)
    full = lambda shape: pl.BlockSpec(shape, lambda i: (0, 0))
    return pl.pallas_call(
        _tail_body,
        grid=(grid,),
        in_specs=[
            pl.BlockSpec((1, block_rows, 128), lambda i: (0, i, 0)),
            pl.BlockSpec((1, block_rows, 128), lambda i: (1, i, 0)),
            pl.BlockSpec((1, block_rows, EXW), lambda i: (0, i, 0)),
            pl.BlockSpec((1, block_rows, EXW), lambda i: (1, i, 0)),
            pl.BlockSpec((block_rows, D), row),
            pl.BlockSpec((block_rows, D), row),
            pl.BlockSpec((block_rows, D), row),
            pl.BlockSpec((block_rows, 3 * H), row),
            full((D, D)), full((1, D)),
            full((D, D)), full((1, D)),
            full((2 * D, D)), full((1, D)),
            full((D, D)), full((1, D)),
            full((D, H)), full((1, H)),
        ],
        out_specs=[pl.BlockSpec((block_rows, D), row),
                   pl.BlockSpec((block_rows, 3 * H), row)],
        out_shape=[jax.ShapeDtypeStruct((N, D), jnp.float32),
                   jax.ShapeDtypeStruct((N, 3 * H), jnp.float32)],
    )(pv, pv, pd, pd, pv0, pv1, prot, pvec12,
      Wh0, bh0.reshape(1, -1), Wh1, bh1.reshape(1, -1),
      Wu0, bu0.reshape(1, -1), Wu1, bu1.reshape(1, -1),
      Wg, bg.reshape(1, -1))


def kernel(source_node_edge_features_exp, prot_scalars, prot_vectors,
           pr_pr_edge_index, lig_scalars, lig_pr_eattr, lig_pr_edge_index,
           W0, a0, V0, W1, a1, V1, Wh0, bh0, Wh1, bh1, Wu0, bu0, Wu1, bu1,
           Wg, bg):
    feats = source_node_edge_features_exp
    N, D = prot_scalars.shape
    DE = feats.shape[1]

    WV0a = jnp.concatenate([W0[:DE], V0[:DE]], axis=1)
    WV1c = jnp.concatenate([W1[2 * D:], V1[2 * D:]], axis=1)
    PN = jnp.concatenate([W0[DE:], V0[DE:], W1[D:2 * D], V1[D:2 * D]], axis=1)
    PP = _mm(prot_scalars, PN, 1000)
    LN = jnp.concatenate([W1[:D], V1[:D]], axis=1)
    LL = _mm(lig_scalars, LN, 1000)

    PW0, PV0, PW1, PV1 = PP[:, :D], PP[:, D:2*D], PP[:, 2*D:3*D], PP[:, 3*D:]
    LW1, LV1 = LL[:, :D], LL[:, D:]

    dst0 = pr_pr_edge_index[1]
    src1 = lig_pr_edge_index[0]
    dst1 = lig_pr_edge_index[1]

    GW0, GSW, GSV, GD = _sc_gather(PW0, dst0, LL, src1, PW1, dst1)
    pay0v, pay0e = _edge0(feats, WV0a, GW0, a0.reshape(1, H * DH))
    pay1v, pay1e = _edge1(lig_pr_eattr, WV1c, GSW, GSV, GD,
                          a1.reshape(1, H * DH))

    pv, pd = _sc_scatter(pay0v, pay0e, dst0, pay1v, pay1e, dst1)

    new_scalars, nv12 = _tail(
        pv, pd, PV0, PV1, prot_scalars,
        prot_vectors.reshape(N, 3 * H),
        Wh0, bh0, Wh1, bh1, Wu0, bu0, Wu1, bu1, Wg, bg)
    return (new_scalars, nv12.reshape(N, H, 3))


# speedup vs baseline: 1.1532x; 1.0049x over previous
import functools

import jax
import jax.numpy as jnp
from jax import lax
from jax.experimental import pallas as pl
from jax.experimental.pallas import tpu as pltpu
from jax.experimental.pallas import tpu_sc as plsc

H = 4
DH = 32
EXW = 16
CH = 96
E_PP = 160000
E_LP = 64000
N_NODES = 10000
N_PAD = 10112
RPT = 632


def _mm_body(x_ref, w_ref, o_ref):
    o_ref[...] = jnp.dot(x_ref[...], w_ref[...],
                         preferred_element_type=jnp.float32)


def _mm(x, w, block_rows):
    M, K = x.shape
    _, N = w.shape
    assert M % block_rows == 0
    return pl.pallas_call(
        _mm_body,
        grid=(M // block_rows,),
        in_specs=[pl.BlockSpec((block_rows, K), lambda i: (i, 0)),
                  pl.BlockSpec((K, N), lambda i: (0, 0))],
        out_specs=pl.BlockSpec((block_rows, N), lambda i: (i, 0)),
        out_shape=jax.ShapeDtypeStruct((M, N), jnp.float32),
    )(x, w)


GCH = 128


def _sc_gather(pw0, dst0, ll, src1, pw1, dst1):
    mesh = plsc.VectorSubcoreMesh(core_axis_name="c", subcore_axis_name="s")

    @functools.partial(
        pl.kernel,
        mesh=mesh,
        compiler_params=pltpu.CompilerParams(use_tc_tiling_on_sc=False),
        out_type=[jax.ShapeDtypeStruct((E_PP, 128), jnp.float32),
                  jax.ShapeDtypeStruct((E_LP, 128), jnp.float32),
                  jax.ShapeDtypeStruct((E_LP, 128), jnp.float32),
                  jax.ShapeDtypeStruct((E_LP, 128), jnp.float32)],
        scratch_types=[
            pltpu.VMEM((GCH,), jnp.int32),
            pltpu.VMEM((GCH,), jnp.int32),
            pltpu.VMEM((GCH, 128), jnp.float32),
            pltpu.VMEM((GCH, 128), jnp.float32),
            pltpu.VMEM((GCH, 256), jnp.float32),
            pltpu.VMEM((GCH, 256), jnp.float32),
            pltpu.SemaphoreType.DMA,
            pltpu.SemaphoreType.DMA,
        ],
    )
    def k(pw0_hbm, dst0_hbm, ll_hbm, src1_hbm, pw1_hbm, dst1_hbm,
          gw0_hbm, gsw_hbm, gsv_hbm, gd_hbm,
          idx_a, idx_b, n_a, n_b, w_a, w_b, sem_a, sem_b):
        c = lax.axis_index("c")
        s = lax.axis_index("s")
        wid = c * 16 + s

        def gseg(idx_hbm, table_hbm, outs, base, nfull, tail, buf_a, buf_b):
            wide = len(outs) > 1

            def wr(buf, o, n):
                for out_hbm, lo in outs:
                    srcb = (buf.at[pl.ds(0, n), pl.ds(lo, 128)]
                            if (wide or n != GCH) else buf)
                    pltpu.sync_copy(srcb, out_hbm.at[pl.ds(o, n)])

            npairs = nfull // 2
            pltpu.sync_copy(idx_hbm.at[pl.ds(base, GCH)], idx_a)
            pltpu.async_copy(table_hbm.at[idx_a], buf_a, sem_a)

            def body(j, _):
                o0 = base + (2 * j) * GCH
                o1 = base + (2 * j + 1) * GCH
                pltpu.sync_copy(idx_hbm.at[pl.ds(o1, GCH)], idx_b)
                pltpu.async_copy(table_hbm.at[idx_b], buf_b, sem_b)
                pltpu.make_async_copy(table_hbm.at[idx_a], buf_a, sem_a).wait()
                wr(buf_a, o0, GCH)

                @pl.when(j < npairs - 1)
                def _():
                    o2 = base + (2 * j + 2) * GCH
                    pltpu.sync_copy(idx_hbm.at[pl.ds(o2, GCH)], idx_a)
                    pltpu.async_copy(table_hbm.at[idx_a], buf_a, sem_a)

                pltpu.make_async_copy(table_hbm.at[idx_b], buf_b, sem_b).wait()
                wr(buf_b, o1, GCH)
                return 0
            lax.fori_loop(0, npairs, body, 0)

            tails = ([(base + 2 * npairs * GCH, GCH)] if nfull % 2 else [])
            tails.append((base + nfull * GCH, tail))
            for t, n in tails:
                it = idx_a.at[pl.ds(0, n)]
                bt = buf_a.at[pl.ds(0, n)]
                pltpu.sync_copy(idx_hbm.at[pl.ds(t, n)], it)
                pltpu.async_copy(table_hbm.at[it], bt, sem_a)
                pltpu.make_async_copy(table_hbm.at[it], bt, sem_a).wait()
                wr(buf_a, t, n)

        base0 = wid * (E_PP // 32)
        base1 = wid * (E_LP // 32)
        gseg(dst0_hbm, pw0_hbm, [(gw0_hbm, 0)], base0, 39, 8, n_a, n_b)
        gseg(src1_hbm, ll_hbm, [(gsw_hbm, 0), (gsv_hbm, 128)],
             base1, 15, 80, w_a, w_b)
        gseg(dst1_hbm, pw1_hbm, [(gd_hbm, 0)], base1, 15, 80, n_a, n_b)

    return k(pw0, dst0, ll, src1, pw1, dst1)


def _iota2(shape, d0, d1, div):
    return (jax.lax.broadcasted_iota(jnp.int32, shape, d0) // div
            == jax.lax.broadcasted_iota(jnp.int32, shape, d1)
            ).astype(jnp.float32)


def _edge0_body(f_ref, w_ref, g_ref, a_ref, o_ref, e_ref):
    x = jnp.dot(f_ref[...], w_ref[...], preferred_element_type=jnp.float32)
    pre = x[:, :128] + g_ref[...]
    hh = jnp.where(pre >= 0, pre, 0.2 * pre)
    sel = _iota2((H * DH, H), 0, 1, DH)
    ex = jnp.exp(jnp.dot(hh * a_ref[...], sel,
                         preferred_element_type=jnp.float32))
    rexp = _iota2((H, H * DH), 1, 0, DH)
    b = ex.shape[0]
    o_ref[...] = jnp.dot(ex, rexp, preferred_element_type=jnp.float32) * x[:, 128:]
    e_ref[...] = jnp.concatenate(
        [ex, jnp.zeros((b, EXW - H), jnp.float32)], axis=1)


def _edge1_body(f_ref, w_ref, gw_ref, gv_ref, gd_ref, a_ref, o_ref, e_ref):
    x = jnp.dot(f_ref[...], w_ref[...], preferred_element_type=jnp.float32)
    pre = x[:, :128] + gw_ref[...] + gd_ref[...]
    fv = x[:, 128:] + gv_ref[...]
    hh = jnp.where(pre >= 0, pre, 0.2 * pre)
    sel = _iota2((H * DH, H), 0, 1, DH)
    ex = jnp.exp(jnp.dot(hh * a_ref[...], sel,
                         preferred_element_type=jnp.float32))
    rexp = _iota2((H, H * DH), 1, 0, DH)
    b = ex.shape[0]
    o_ref[...] = jnp.dot(ex, rexp, preferred_element_type=jnp.float32) * fv
    e_ref[...] = jnp.concatenate(
        [jnp.zeros((b, H), jnp.float32), ex,
         jnp.zeros((b, EXW - 2 * H), jnp.float32)], axis=1)


def _edge0(f, w, g, a0f, block_rows=8000):
    E, K = f.shape
    row = lambda i: (i, 0)
    return pl.pallas_call(
        _edge0_body,
        grid=(E // block_rows,),
        in_specs=[pl.BlockSpec((block_rows, K), row),
                  pl.BlockSpec((K, 256), lambda i: (0, 0)),
                  pl.BlockSpec((block_rows, 128), row),
                  pl.BlockSpec((1, 128), lambda i: (0, 0))],
        out_specs=[pl.BlockSpec((block_rows, 128), row),
                   pl.BlockSpec((block_rows, EXW), row)],
        out_shape=[jax.ShapeDtypeStruct((E, 128), jnp.float32),
                   jax.ShapeDtypeStruct((E, EXW), jnp.float32)],
    )(f, w, g, a0f)


def _edge1(f, w, gw, gv, gd, a1f, block_rows=8000):
    E, K = f.shape
    row = lambda i: (i, 0)
    return pl.pallas_call(
        _edge1_body,
        grid=(E // block_rows,),
        in_specs=[pl.BlockSpec((block_rows, K), row),
                  pl.BlockSpec((K, 256), lambda i: (0, 0)),
                  pl.BlockSpec((block_rows, 128), row),
                  pl.BlockSpec((block_rows, 128), row),
                  pl.BlockSpec((block_rows, 128), row),
                  pl.BlockSpec((1, 128), lambda i: (0, 0))],
        out_specs=[pl.BlockSpec((block_rows, 128), row),
                   pl.BlockSpec((block_rows, EXW), row)],
        out_shape=[jax.ShapeDtypeStruct((E, 128), jnp.float32),
                   jax.ShapeDtypeStruct((E, EXW), jnp.float32)],
    )(f, w, gw, gv, gd, a1f)


def _sc_scatter(pv0, pe0, idx0, pv1, pe1, idx1):
    mesh = plsc.VectorSubcoreMesh(core_axis_name="c", subcore_axis_name="s")

    @functools.partial(
        pl.kernel,
        mesh=mesh,
        compiler_params=pltpu.CompilerParams(use_tc_tiling_on_sc=False),
        out_type=[jax.ShapeDtypeStruct((2, N_PAD, 128), jnp.float32),
                  jax.ShapeDtypeStruct((2, N_PAD, EXW), jnp.float32)],
        scratch_types=[
            pltpu.VMEM_SHARED((N_PAD, 128), jnp.float32),
            pltpu.VMEM_SHARED((N_PAD, EXW), jnp.float32),
            pltpu.VMEM((16, 128), jnp.float32),
            pltpu.VMEM((16, EXW), jnp.float32),
            pltpu.VMEM((CH,), jnp.int32),
            pltpu.VMEM((CH,), jnp.int32),
            pltpu.VMEM((CH, 128), jnp.float32),
            pltpu.VMEM((CH, 128), jnp.float32),
            pltpu.VMEM((CH, EXW), jnp.float32),
            pltpu.VMEM((CH, EXW), jnp.float32),
            pltpu.VMEM((8,), jnp.int32),
            pltpu.VMEM((80,), jnp.int32),
            pltpu.SemaphoreType.DMA,
            pltpu.SemaphoreType.DMA,
        ],
    )
    def k(pv0_hbm, pe0_hbm, idx0_hbm, pv1_hbm, pe1_hbm, idx1_hbm,
          outv_hbm, outd_hbm,
          accv, accd, zerov, zerod, idx_a, idx_b, pv_a, pv_b, pe_a, pe_b,
          idx_t0, idx_t1, sem_a, sem_b):
        c = lax.axis_index("c")
        s = lax.axis_index("s")
        wid = c * 16 + s

        for i in range(16):
            for j in range(128 // 16):
                zerov[i, pl.ds(j * 16, 16)] = jnp.zeros((16,), jnp.float32)
            zerod[i, pl.ds(0, EXW)] = jnp.zeros((EXW,), jnp.float32)
        row0 = s * RPT

        def zbody(r, _):
            pltpu.sync_copy(zerov, accv.at[pl.ds(row0 + r * 16, 16)])
            pltpu.sync_copy(zerod, accd.at[pl.ds(row0 + r * 16, 16)])
            return 0
        lax.fori_loop(0, RPT // 16, zbody, 0)
        pltpu.sync_copy(zerov.at[pl.ds(0, 8)],
                        accv.at[pl.ds(row0 + 16 * (RPT // 16), 8)])
        pltpu.sync_copy(zerod.at[pl.ds(0, 8)],
                        accd.at[pl.ds(row0 + 16 * (RPT // 16), 8)])
        plsc.subcore_barrier()

        def seg(pv_hbm, pe_hbm, idx_hbm, base, npairs):
            pltpu.async_copy(idx_hbm.at[pl.ds(base, CH)], idx_a, sem_a)
            pltpu.async_copy(pv_hbm.at[pl.ds(base, CH)], pv_a, sem_a)
            pltpu.async_copy(pe_hbm.at[pl.ds(base, CH)], pe_a, sem_a)

            def body(j, _):
                o1 = base + (2 * j + 1) * CH
                pltpu.async_copy(idx_hbm.at[pl.ds(o1, CH)], idx_b, sem_b)
                pltpu.async_copy(pv_hbm.at[pl.ds(o1, CH)], pv_b, sem_b)
                pltpu.async_copy(pe_hbm.at[pl.ds(o1, CH)], pe_b, sem_b)
                pltpu.make_async_copy(idx_hbm.at[pl.ds(base, CH)], idx_a,
                                      sem_a).wait()
                pltpu.make_async_copy(pv_hbm.at[pl.ds(base, CH)], pv_a,
                                      sem_a).wait()
                pltpu.make_async_copy(pe_hbm.at[pl.ds(base, CH)], pe_a,
                                      sem_a).wait()
                pltpu.sync_copy(pv_a, accv.at[idx_a], add=True)
                pltpu.sync_copy(pe_a, accd.at[idx_a], add=True)

                @pl.when(j < npairs - 1)
                def _():
                    o2 = base + (2 * j + 2) * CH
                    pltpu.async_copy(idx_hbm.at[pl.ds(o2, CH)], idx_a, sem_a)
                    pltpu.async_copy(pv_hbm.at[pl.ds(o2, CH)], pv_a, sem_a)
                    pltpu.async_copy(pe_hbm.at[pl.ds(o2, CH)], pe_a, sem_a)

                pltpu.make_async_copy(idx_hbm.at[pl.ds(o1, CH)], idx_b,
                                      sem_b).wait()
                pltpu.make_async_copy(pv_hbm.at[pl.ds(o1, CH)], pv_b,
                                      sem_b).wait()
                pltpu.make_async_copy(pe_hbm.at[pl.ds(o1, CH)], pe_b,
                                      sem_b).wait()
                pltpu.sync_copy(pv_b, accv.at[idx_b], add=True)
                pltpu.sync_copy(pe_b, accd.at[idx_b], add=True)
                return 0
            lax.fori_loop(0, npairs, body, 0)

        base0 = wid * (E_PP // 32)
        seg(pv0_hbm, pe0_hbm, idx0_hbm, base0, 26)
        t0 = base0 + 52 * CH
        pltpu.sync_copy(idx0_hbm.at[pl.ds(t0, 8)], idx_t0)
        pltpu.sync_copy(pv0_hbm.at[pl.ds(t0, 8)], pv_a.at[pl.ds(0, 8)])
        pltpu.sync_copy(pe0_hbm.at[pl.ds(t0, 8)], pe_a.at[pl.ds(0, 8)])
        pltpu.sync_copy(pv_a.at[pl.ds(0, 8)], accv.at[idx_t0], add=True)
        pltpu.sync_copy(pe_a.at[pl.ds(0, 8)], accd.at[idx_t0], add=True)

        base1 = wid * (E_LP // 32)
        seg(pv1_hbm, pe1_hbm, idx1_hbm, base1, 10)
        t1 = base1 + 20 * CH
        pltpu.sync_copy(idx1_hbm.at[pl.ds(t1, 80)], idx_t1)
        pltpu.sync_copy(pv1_hbm.at[pl.ds(t1, 80)], pv_a.at[pl.ds(0, 80)])
        pltpu.sync_copy(pe1_hbm.at[pl.ds(t1, 80)], pe_a.at[pl.ds(0, 80)])
        pltpu.sync_copy(pv_a.at[pl.ds(0, 80)], accv.at[idx_t1], add=True)
        pltpu.sync_copy(pe_a.at[pl.ds(0, 80)], accd.at[idx_t1], add=True)

        plsc.subcore_barrier()
        pltpu.sync_copy(accv.at[pl.ds(row0, RPT)],
                        outv_hbm.at[c, pl.ds(row0, RPT)])
        pltpu.sync_copy(accd.at[pl.ds(row0, RPT)],
                        outd_hbm.at[c, pl.ds(row0, RPT)])

    return k(pv0, pe0, idx0, pv1, pe1, idx1)


def _head_expand(x):
    r = (jax.lax.broadcasted_iota(jnp.int32, (H, H * DH), 1) // DH
         == jax.lax.broadcasted_iota(jnp.int32, (H, H * DH), 0)
         ).astype(jnp.float32)
    return jnp.dot(x, r, preferred_element_type=jnp.float32)


def _tail_body(v0_ref, v1_ref, d0_ref, d1_ref, pv0_ref, pv1_ref, prot_ref,
               pvec_ref, wh0_ref, bh0_ref, wh1_ref, bh1_ref, wu0_ref,
               bu0_ref, wu1_ref, bu1_ref, wg_ref, bg_ref,
               os_ref, ov_ref):
    t = v0_ref[0] + v1_ref[0]
    d = d0_ref[0] + d1_ref[0]
    den0 = d[:, :H]
    den1 = d[:, H:2 * H]
    numf = (t + _head_expand(den0) * pv0_ref[...]
            + _head_expand(den1) * pv1_ref[...])
    agg = numf / (_head_expand(den0 + den1) + 1e-9)
    h = jax.nn.gelu(jnp.dot(agg, wh0_ref[...],
                            preferred_element_type=jnp.float32) + bh0_ref[...])
    h = jax.nn.gelu(jnp.dot(h, wh1_ref[...],
                            preferred_element_type=jnp.float32) + bh1_ref[...])
    prot = prot_ref[...]
    u = jax.nn.gelu(jnp.dot(jnp.concatenate([prot, h], axis=-1), wu0_ref[...],
                            preferred_element_type=jnp.float32) + bu0_ref[...])
    u = jnp.dot(u, wu1_ref[...], preferred_element_type=jnp.float32) + bu1_ref[...]
    ns = prot + u
    g = jax.nn.sigmoid(jnp.dot(ns, wg_ref[...],
                               preferred_element_type=jnp.float32) + bg_ref[...])
    r3 = (jax.lax.broadcasted_iota(jnp.int32, (H, 3 * H), 1) // 3
          == jax.lax.broadcasted_iota(jnp.int32, (H, 3 * H), 0)
          ).astype(jnp.float32)
    gexp = jnp.dot(g, r3, preferred_element_type=jnp.float32)
    os_ref[...] = ns
    ov_ref[...] = pvec_ref[...] * gexp


def _tail(pv, pd, pv0, pv1, prot, pvec12,
          Wh0, bh0, Wh1, bh1, Wu0, bu0, Wu1, bu1, Wg, bg, block_rows=2000):
    N = prot.shape[0]
    D = prot.shape[1]
    grid = N // block_rows
    row = lambda i: (i, 0)
    full = lambda shape: pl.BlockSpec(shape, lambda i: (0, 0))
    return pl.pallas_call(
        _tail_body,
        grid=(grid,),
        in_specs=[
            pl.BlockSpec((1, block_rows, 128), lambda i: (0, i, 0)),
            pl.BlockSpec((1, block_rows, 128), lambda i: (1, i, 0)),
            pl.BlockSpec((1, block_rows, EXW), lambda i: (0, i, 0)),
            pl.BlockSpec((1, block_rows, EXW), lambda i: (1, i, 0)),
            pl.BlockSpec((block_rows, D), row),
            pl.BlockSpec((block_rows, D), row),
            pl.BlockSpec((block_rows, D), row),
            pl.BlockSpec((block_rows, 3 * H), row),
            full((D, D)), full((1, D)),
            full((D, D)), full((1, D)),
            full((2 * D, D)), full((1, D)),
            full((D, D)), full((1, D)),
            full((D, H)), full((1, H)),
        ],
        out_specs=[pl.BlockSpec((block_rows, D), row),
                   pl.BlockSpec((block_rows, 3 * H), row)],
        out_shape=[jax.ShapeDtypeStruct((N, D), jnp.float32),
                   jax.ShapeDtypeStruct((N, 3 * H), jnp.float32)],
    )(pv, pv, pd, pd, pv0, pv1, prot, pvec12,
      Wh0, bh0.reshape(1, -1), Wh1, bh1.reshape(1, -1),
      Wu0, bu0.reshape(1, -1), Wu1, bu1.reshape(1, -1),
      Wg, bg.reshape(1, -1))


def kernel(source_node_edge_features_exp, prot_scalars, prot_vectors,
           pr_pr_edge_index, lig_scalars, lig_pr_eattr, lig_pr_edge_index,
           W0, a0, V0, W1, a1, V1, Wh0, bh0, Wh1, bh1, Wu0, bu0, Wu1, bu1,
           Wg, bg):
    feats = source_node_edge_features_exp
    N, D = prot_scalars.shape
    DE = feats.shape[1]

    WV0a = jnp.concatenate([W0[:DE], V0[:DE]], axis=1)
    WV1c = jnp.concatenate([W1[2 * D:], V1[2 * D:]], axis=1)
    PN = jnp.concatenate([W0[DE:], V0[DE:], W1[D:2 * D], V1[D:2 * D]], axis=1)
    PP = _mm(prot_scalars, PN, 1000)
    LN = jnp.concatenate([W1[:D], V1[:D]], axis=1)
    LL = _mm(lig_scalars, LN, 1000)

    PW0, PV0, PW1, PV1 = PP[:, :D], PP[:, D:2*D], PP[:, 2*D:3*D], PP[:, 3*D:]
    LW1, LV1 = LL[:, :D], LL[:, D:]

    dst0 = pr_pr_edge_index[1]
    src1 = lig_pr_edge_index[0]
    dst1 = lig_pr_edge_index[1]

    GW0, GSW, GSV, GD = _sc_gather(PW0, dst0, LL, src1, PW1, dst1)
    pay0v, pay0e = _edge0(feats, WV0a, GW0, a0.reshape(1, H * DH))
    pay1v, pay1e = _edge1(lig_pr_eattr, WV1c, GSW, GSV, GD,
                          a1.reshape(1, H * DH))

    pv, pd = _sc_scatter(pay0v, pay0e, dst0, pay1v, pay1e, dst1)

    new_scalars, nv12 = _tail(
        pv, pd, PV0, PV1, prot_scalars,
        prot_vectors.reshape(N, 3 * H),
        Wh0, bh0, Wh1, bh1, Wu0, bu0, Wu1, bu1, Wg, bg)
    return (new_scalars, nv12.reshape(N, H, 3))
